# Initial kernel scaffold; baseline (speedup 1.0000x reference)
import jax
import jax.numpy as jnp
from jax.experimental import pallas as pl


def kernel(cosine, label):
    raise NotImplementedError("write your pallas kernel here")


# speedup vs baseline: 2.7480x; 2.7480x over previous
import functools
import math

import jax
import jax.numpy as jnp
from jax.experimental import pallas as pl
from jax.experimental.pallas import tpu as pltpu

_S = 64.0
_M = 0.5
_COS_M = math.cos(_M)
_SIN_M = math.sin(_M)
_TH = math.cos(math.pi - _M)
_MM = math.sin(math.pi - _M) * _M

_NEG_INF = float("-inf")


def _loss_kernel(cos_ref, lab_ref, out_ref, m_ref, s_ref, cl_ref, *, blk_c, n_blk, n_cols):
    j = pl.program_id(0)

    @pl.when(j == 0)
    def _init():
        m_ref[...] = jnp.full_like(m_ref, _NEG_INF)
        s_ref[...] = jnp.zeros_like(s_ref)
        cl_ref[...] = jnp.full_like(cl_ref, _NEG_INF)

    x = cos_ref[...]
    shape = x.shape
    col = j * blk_c + jax.lax.broadcasted_iota(jnp.int32, shape, 1)
    xs = jnp.where(col < n_cols, x * _S, _NEG_INF)

    is_lab = col == lab_ref[...]
    cl_blk = jnp.max(jnp.where(is_lab, x, _NEG_INF), axis=1, keepdims=True)
    cl_ref[...] = jnp.maximum(cl_ref[...], cl_blk)

    m_old = m_ref[...]
    m_new = jnp.maximum(m_old, jnp.max(xs, axis=1, keepdims=True))
    s_ref[...] = s_ref[...] * jnp.exp(m_old - m_new) + jnp.sum(
        jnp.exp(xs - m_new), axis=1, keepdims=True
    )
    m_ref[...] = m_new

    @pl.when(j == n_blk - 1)
    def _epilogue():
        c_l = cl_ref[...]
        m = m_ref[...]
        s = s_ref[...]
        sine = jnp.sqrt(jnp.clip(1.0 - c_l * c_l, 0.0, 1.0))
        phi = c_l * _COS_M - sine * _SIN_M
        phi = jnp.where(c_l > _TH, phi, c_l - _MM)
        exp_cl = jnp.exp(_S * c_l - m)
        exp_phi = jnp.exp(_S * phi - m)
        s_adj = jnp.maximum(s - exp_cl, 0.0) + exp_phi
        loss = m + jnp.log(s_adj) - _S * phi
        out_ref[0, 0] = jnp.sum(loss) / loss.shape[0]


def kernel(cosine, label):
    b, c = cosine.shape
    blk_c = 2048
    n_blk = pl.cdiv(c, blk_c)
    lab = label.astype(jnp.int32).reshape(b, 1)

    out = pl.pallas_call(
        functools.partial(_loss_kernel, blk_c=blk_c, n_blk=n_blk, n_cols=c),
        grid=(n_blk,),
        in_specs=[
            pl.BlockSpec((b, blk_c), lambda j: (0, j)),
            pl.BlockSpec((b, 1), lambda j: (0, 0)),
        ],
        out_specs=pl.BlockSpec(memory_space=pltpu.SMEM),
        out_shape=jax.ShapeDtypeStruct((1, 1), jnp.float32),
        scratch_shapes=[
            pltpu.VMEM((b, 1), jnp.float32),
            pltpu.VMEM((b, 1), jnp.float32),
            pltpu.VMEM((b, 1), jnp.float32),
        ],
    )(cosine, lab)
    return out[0, 0]
